# baseline (speedup 1.0000x reference)
import jax
import jax.numpy as jnp
from jax.experimental import pallas as pl
from jax.experimental.pallas import tpu as pltpu

BN_EPS = 1e-5


def _stream_kernel(body_ref, face_ref, a_ref, out_ref):
    body = body_ref[...]
    out_ref[0] = jnp.mean(body, axis=(1, 2))
    prod = body * (a_ref[...] * face_ref[...])
    out_ref[1] = jnp.max(prod, axis=(1, 2))


def _make_manual_stream(nseq, tb, nbuf):
    def _manual_stream(body_hbm, face_hbm, a_ref, out_ref,
                       bbuf, fbuf, bsem, fsem):
        core = pl.program_id(0)
        base = core * nseq

        def start(k, slot):
            row = (base + k) * tb
            pltpu.make_async_copy(
                body_hbm.at[pl.ds(row, tb)], bbuf.at[slot], bsem.at[slot]).start()
            pltpu.make_async_copy(
                face_hbm.at[pl.ds(row, tb)], fbuf.at[slot], fsem.at[slot]).start()

        for p in range(min(nbuf, nseq)):
            start(p, p)
        for k in range(nseq):
            slot = k % nbuf
            pltpu.make_async_copy(
                bbuf.at[slot], bbuf.at[slot], bsem.at[slot]).wait()
            pltpu.make_async_copy(
                fbuf.at[slot], fbuf.at[slot], fsem.at[slot]).wait()
            body = bbuf[slot]
            out_ref[0, pl.ds(k * tb, tb)] = jnp.mean(body, axis=(1, 2))
            prod = body * (a_ref[...] * fbuf[slot])
            out_ref[1, pl.ds(k * tb, tb)] = jnp.max(prod, axis=(1, 2))
            if k + nbuf < nseq:
                start(k + nbuf, slot)

    return _manual_stream


def _head_kernel(meanmx_ref, gate_ref, w1_ref, b1_ref, w2_ref, b2_ref,
                 wl_ref, bl_ref, gamma_ref, beta_ref, out_ref):
    h = jnp.dot(meanmx_ref[1], w1_ref[...],
                preferred_element_type=jnp.float32) + b1_ref[...]
    h = jnp.maximum(h, 0.0)
    s = jax.nn.sigmoid(
        jnp.dot(h, w2_ref[...], preferred_element_type=jnp.float32) + b2_ref[...])
    feat = meanmx_ref[0] + gate_ref[...] * s
    y = jnp.dot(feat, wl_ref[...],
                preferred_element_type=jnp.float32) + bl_ref[...]
    mu = jnp.mean(y, axis=0, keepdims=True)
    var = jnp.mean(jnp.square(y - mu), axis=0, keepdims=True)
    out_ref[...] = (gamma_ref[...] * (y - mu) * jax.lax.rsqrt(var + BN_EPS)
                    + beta_ref[...])


def kernel(x_body, x_face, pose, A_front, a_hwc, w1_t, b1, w2_t, b2, wl_t, bl, gamma, beta):
    B, H, W, C = x_body.shape
    HID = w1_t.shape[1]
    nattr = wl_t.shape[1]

    body = jnp.transpose(x_body, (0, 2, 1, 3))
    face = jnp.transpose(x_face, (0, 2, 1, 3))
    a_4d = jnp.transpose(A_front, (2, 1, 0))

    TB = 16
    B_pad = ((B + TB - 1) // TB) * TB
    if B_pad != B:
        pad = B_pad - B
        body = jnp.pad(body, ((0, pad), (0, 0), (0, 0), (0, 0)))
        face = jnp.pad(face, ((0, pad), (0, 0), (0, 0), (0, 0)))
    nblk = B_pad // TB

    if nblk % 2 == 0:
        nseq = nblk // 2
        NBUF = 3
        meanmx = pl.pallas_call(
            _make_manual_stream(nseq, TB, NBUF),
            out_shape=jax.ShapeDtypeStruct((2, B_pad, C), jnp.float32),
            grid=(2,),
            in_specs=[
                pl.BlockSpec(memory_space=pl.ANY),
                pl.BlockSpec(memory_space=pl.ANY),
                pl.BlockSpec((W, H, C), lambda i: (0, 0, 0)),
            ],
            out_specs=pl.BlockSpec((2, B_pad // 2, C), lambda i: (0, i, 0)),
            scratch_shapes=[
                pltpu.VMEM((NBUF, TB, W, H, C), jnp.float32),
                pltpu.VMEM((NBUF, TB, W, H, C), jnp.float32),
                pltpu.SemaphoreType.DMA((NBUF,)),
                pltpu.SemaphoreType.DMA((NBUF,)),
            ],
            compiler_params=pltpu.CompilerParams(
                dimension_semantics=("parallel",),
                vmem_limit_bytes=60 * 1024 * 1024,
            ),
        )(body, face, a_4d)
    else:
        meanmx = pl.pallas_call(
            _stream_kernel,
            out_shape=jax.ShapeDtypeStruct((2, B_pad, C), jnp.float32),
            grid=(nblk,),
            in_specs=[
                pl.BlockSpec((TB, W, H, C), lambda i: (i, 0, 0, 0)),
                pl.BlockSpec((TB, W, H, C), lambda i: (i, 0, 0, 0)),
                pl.BlockSpec((W, H, C), lambda i: (0, 0, 0)),
            ],
            out_specs=pl.BlockSpec((2, TB, C), lambda i: (0, i, 0)),
            compiler_params=pltpu.CompilerParams(
                dimension_semantics=("parallel",),
                vmem_limit_bytes=60 * 1024 * 1024,
            ),
        )(body, face, a_4d)

    if B_pad != B:
        meanmx = meanmx[:, :B]

    gate = (pose.astype(jnp.int32) == 1).astype(jnp.float32).reshape(B, 1)

    nattr_pad = ((nattr + 127) // 128) * 128
    pad_n = nattr_pad - nattr
    if pad_n:
        wl_t = jnp.pad(wl_t, ((0, 0), (0, pad_n)))
        bl = jnp.pad(bl, ((0, 0), (0, pad_n)))
        gamma = jnp.pad(gamma, ((0, 0), (0, pad_n)))
        beta = jnp.pad(beta, ((0, 0), (0, pad_n)))

    out = pl.pallas_call(
        _head_kernel,
        out_shape=jax.ShapeDtypeStruct((B, nattr_pad), jnp.float32),
        grid=(1,),
        in_specs=[
            pl.BlockSpec((2, B, C), lambda j: (0, 0, 0)),
            pl.BlockSpec((B, 1), lambda j: (0, 0)),
            pl.BlockSpec((C, HID), lambda j: (0, 0)),
            pl.BlockSpec((1, HID), lambda j: (0, 0)),
            pl.BlockSpec((HID, C), lambda j: (0, 0)),
            pl.BlockSpec((1, C), lambda j: (0, 0)),
            pl.BlockSpec((C, nattr_pad), lambda j: (0, 0)),
            pl.BlockSpec((1, nattr_pad), lambda j: (0, 0)),
            pl.BlockSpec((1, nattr_pad), lambda j: (0, 0)),
            pl.BlockSpec((1, nattr_pad), lambda j: (0, 0)),
        ],
        out_specs=pl.BlockSpec((B, nattr_pad), lambda j: (0, 0)),
        compiler_params=pltpu.CompilerParams(
            dimension_semantics=("arbitrary",),
            vmem_limit_bytes=48 * 1024 * 1024,
        ),
    )(meanmx, gate, w1_t, b1, w2_t, b2, wl_t, bl, gamma, beta)
    return out[:, :nattr]


# speedup vs baseline: 1.0571x; 1.0571x over previous
import jax
import jax.numpy as jnp
from jax.experimental import pallas as pl
from jax.experimental.pallas import tpu as pltpu

BN_EPS = 1e-5


def _stream_kernel(body_ref, face_ref, a_ref, out_ref):
    body = body_ref[...]
    out_ref[0] = jnp.mean(body, axis=(1, 2))
    prod = body * (a_ref[...] * face_ref[...])
    out_ref[1] = jnp.max(prod, axis=(1, 2))


def _head_kernel(meanmx_ref, gate_ref, w1_ref, b1_ref, w2_ref, b2_ref,
                 wl_ref, bl_ref, gamma_ref, beta_ref, out_ref):
    h = jnp.dot(meanmx_ref[1], w1_ref[...],
                preferred_element_type=jnp.float32) + b1_ref[...]
    h = jnp.maximum(h, 0.0)
    s = jax.nn.sigmoid(
        jnp.dot(h, w2_ref[...], preferred_element_type=jnp.float32) + b2_ref[...])
    feat = meanmx_ref[0] + gate_ref[...] * s
    y = jnp.dot(feat, wl_ref[...],
                preferred_element_type=jnp.float32) + bl_ref[...]
    mu = jnp.mean(y, axis=0, keepdims=True)
    var = jnp.mean(jnp.square(y - mu), axis=0, keepdims=True)
    out_ref[...] = (gamma_ref[...] * (y - mu) * jax.lax.rsqrt(var + BN_EPS)
                    + beta_ref[...])


def kernel(x_body, x_face, pose, A_front, a_hwc, w1_t, b1, w2_t, b2, wl_t, bl, gamma, beta):
    B, H, W, C = x_body.shape
    HID = w1_t.shape[1]
    nattr = wl_t.shape[1]

    body = jnp.transpose(x_body, (0, 2, 1, 3))
    face = jnp.transpose(x_face, (0, 2, 1, 3))
    a_4d = jnp.transpose(A_front, (2, 1, 0))

    TB = 16
    B_pad = ((B + TB - 1) // TB) * TB
    if B_pad != B:
        pad = B_pad - B
        body = jnp.pad(body, ((0, pad), (0, 0), (0, 0), (0, 0)))
        face = jnp.pad(face, ((0, pad), (0, 0), (0, 0), (0, 0)))
    nblk = B_pad // TB

    meanmx = pl.pallas_call(
        _stream_kernel,
        out_shape=jax.ShapeDtypeStruct((2, B_pad, C), jnp.float32),
        grid=(nblk,),
        in_specs=[
            pl.BlockSpec((TB, W, H, C), lambda i: (i, 0, 0, 0)),
            pl.BlockSpec((TB, W, H, C), lambda i: (i, 0, 0, 0)),
            pl.BlockSpec((W, H, C), lambda i: (0, 0, 0)),
        ],
        out_specs=pl.BlockSpec((2, TB, C), lambda i: (0, i, 0)),
        compiler_params=pltpu.CompilerParams(
            dimension_semantics=("parallel",),
            vmem_limit_bytes=60 * 1024 * 1024,
        ),
    )(body, face, a_4d)

    if B_pad != B:
        meanmx = meanmx[:, :B]

    gate = (pose.astype(jnp.int32) == 1).astype(jnp.float32).reshape(B, 1)

    nattr_pad = ((nattr + 127) // 128) * 128
    pad_n = nattr_pad - nattr
    if pad_n:
        wl_t = jnp.pad(wl_t, ((0, 0), (0, pad_n)))
        bl = jnp.pad(bl, ((0, 0), (0, pad_n)))
        gamma = jnp.pad(gamma, ((0, 0), (0, pad_n)))
        beta = jnp.pad(beta, ((0, 0), (0, pad_n)))

    out = pl.pallas_call(
        _head_kernel,
        out_shape=jax.ShapeDtypeStruct((B, nattr_pad), jnp.float32),
        grid=(1,),
        in_specs=[
            pl.BlockSpec((2, B, C), lambda j: (0, 0, 0)),
            pl.BlockSpec((B, 1), lambda j: (0, 0)),
            pl.BlockSpec((C, HID), lambda j: (0, 0)),
            pl.BlockSpec((1, HID), lambda j: (0, 0)),
            pl.BlockSpec((HID, C), lambda j: (0, 0)),
            pl.BlockSpec((1, C), lambda j: (0, 0)),
            pl.BlockSpec((C, nattr_pad), lambda j: (0, 0)),
            pl.BlockSpec((1, nattr_pad), lambda j: (0, 0)),
            pl.BlockSpec((1, nattr_pad), lambda j: (0, 0)),
            pl.BlockSpec((1, nattr_pad), lambda j: (0, 0)),
        ],
        out_specs=pl.BlockSpec((B, nattr_pad), lambda j: (0, 0)),
        compiler_params=pltpu.CompilerParams(
            dimension_semantics=("arbitrary",),
            vmem_limit_bytes=48 * 1024 * 1024,
        ),
    )(meanmx, gate, w1_t, b1, w2_t, b2, wl_t, bl, gamma, beta)
    return out[:, :nattr]
